# Initial kernel scaffold; baseline (speedup 1.0000x reference)
import jax
import jax.numpy as jnp
from jax.experimental import pallas as pl


def kernel(x, edge_attr, mol_attr, mlogP, volumn, wt, params, edge_index, batch):
    raise NotImplementedError("write your pallas kernel here")


# speedup vs baseline: 12.5033x; 12.5033x over previous
import functools
import jax
import jax.numpy as jnp
from jax import lax
from jax.experimental import pallas as pl
from jax.experimental.pallas import tpu as pltpu
from jax.experimental.pallas import tpu_sc as plsc

_N = 10000
_E = 320000
_B = 64
_D = 128
_F32 = jnp.float32

_NC, _NS = 2, 16
_NW = _NC * _NS
_EB = 128
_NBLK = _E // _EB
_RT = _N // _NS
_RS = _N // 10
_CHUNKS = (0, 128, 256, 384, 512, 640, 768, 872)


def _leaky(x):
    return jnp.where(x >= 0, x, 0.01 * x)


def _allsum16(x):
    lane = lax.broadcasted_iota(jnp.int32, (16,), 0)
    for k in (8, 4, 2, 1):
        perm = jnp.bitwise_xor(lane, k)
        x = x + x.at[perm].get(mode="promise_in_bounds")
    return x


def _elu(x):
    return jnp.where(x > 0, x, jnp.exp(jnp.minimum(x, 0.0)) - 1.0)


def _dot(a, b):
    return jnp.dot(a, b, preferred_element_type=_F32)


def _dot_hi(a, b):
    return jnp.dot(a, b, preferred_element_type=_F32,
                   precision=lax.Precision.HIGHEST)


def _gru(xin, h, wih_t, whh_t, bih, bhh):
    gi = _dot(xin, wih_t) + bih
    gh = _dot(h, whh_t) + bhh
    r = jax.nn.sigmoid(gi[:, :_D] + gh[:, :_D])
    z = jax.nn.sigmoid(gi[:, _D:2 * _D] + gh[:, _D:2 * _D])
    n = jnp.tanh(gi[:, 2 * _D:] + r * gh[:, 2 * _D:])
    return (1.0 - z) * n + z * h


def _tc_nodes_body(x, wn_t, bn, wl1_t, bl1, a_t, g2_t, att_r,
                   h0_o, u_o, w_o, r_o):
    xe = jnp.maximum(_dot(x[...], wn_t[...]) + bn[...], 0.0)
    h0 = _leaky(_dot(xe, wl1_t[...]) + bl1[...])
    h0_o[...] = h0
    u_o[...] = _dot(h0, a_t[...])
    w_o[...] = _dot(h0, g2_t[...])
    r_o[...] = jnp.sum(h0 * att_r[...][None, :], axis=1, keepdims=True)


def _tc_nodes(x, wn_t, bn, wl1_t, bl1, a_t, g2_t, att_r):
    nb = 10
    rows = _N // nb
    full = lambda s: pl.BlockSpec(s, lambda i: (0,) * len(s))
    return pl.pallas_call(
        _tc_nodes_body,
        grid=(nb,),
        in_specs=[
            pl.BlockSpec((rows, _D), lambda i: (i, 0)),
            full((_D, _D)), full((_D,)), full((_D, _D)), full((_D,)),
            full((_D, _D)), full((_D, _D)), full((_D,)),
        ],
        out_specs=[
            pl.BlockSpec((rows, _D), lambda i: (i, 0)),
            pl.BlockSpec((rows, _D), lambda i: (i, 0)),
            pl.BlockSpec((rows, _D), lambda i: (i, 0)),
            pl.BlockSpec((rows, 1), lambda i: (i, 0)),
        ],
        out_shape=[
            jax.ShapeDtypeStruct((_N, _D), _F32),
            jax.ShapeDtypeStruct((_N, _D), _F32),
            jax.ShapeDtypeStruct((_N, _D), _F32),
            jax.ShapeDtypeStruct((_N, 1), _F32),
        ],
    )(x, wn_t, bn, wl1_t, bl1, a_t, g2_t, att_r)


def _tc_edges_body(ea, we_t, be, b_t, v_o):
    ee = jnp.maximum(_dot(ea[...], we_t[...]) + be[...], 0.0)
    v_o[...] = _dot(ee, b_t[...])


def _tc_edges(edge_attr, we_t, be, b_t):
    nb = 160
    rows = _E // nb
    full = lambda s: pl.BlockSpec(s, lambda i: (0,) * len(s))
    return pl.pallas_call(
        _tc_edges_body,
        grid=(nb,),
        in_specs=[
            pl.BlockSpec((rows, 16), lambda i: (i, 0)),
            full((16, 16)), full((16,)), full((16, _D)),
        ],
        out_specs=pl.BlockSpec((rows, _D), lambda i: (i, 0)),
        out_shape=jax.ShapeDtypeStruct((_E, _D), _F32),
    )(edge_attr, we_t, be, b_t)


def _sc_gate_body(u_hbm, v_hbm, w_hbm, r_hbm, attl_hbm, src_hbm, dst_hbm,
                  acc_o, s_o,
                  srcb, dstb, ub, vb, eb, rloc, attl_v,
                  acc_sh, s_sh):
    c = lax.axis_index("c")
    s = lax.axis_index("s")
    wid = s * _NC + c
    def zrow(j, c2):
        for q in range(8):
            ub[j, pl.ds(q * 16, 16)] = jnp.zeros((16,), _F32)
        return c2

    lax.fori_loop(0, _EB, zrow, 0)
    for q in range(8):
        eb[pl.ds(q * 16, 16)] = jnp.zeros((16,), _F32)
    for k in range(5):
        pltpu.sync_copy(ub.at[pl.ds(0, 125)],
                        acc_sh.at[pl.ds(s * 625 + k * 125, 125)])

    @pl.when(s < 10)
    def _():
        for k in _CHUNKS:
            pltpu.sync_copy(eb, s_sh.at[pl.ds(s * _RS + k, _EB)])

    pltpu.sync_copy(r_hbm, rloc)
    pltpu.sync_copy(attl_hbm, attl_v)
    plsc.subcore_barrier()

    nblk = _NBLK // _NW + jnp.where(wid < _NBLK % _NW, 1, 0)

    def blk(i, carry):
        base = (wid + i * _NW) * _EB
        base = pl.multiple_of(base, _EB)
        pltpu.sync_copy(src_hbm.at[pl.ds(base, _EB)], srcb)
        pltpu.sync_copy(dst_hbm.at[pl.ds(base, _EB)], dstb)
        pltpu.sync_copy(v_hbm.at[pl.ds(base, _EB)], vb)
        pltpu.sync_copy(u_hbm.at[srcb], ub)

        lane = lax.broadcasted_iota(jnp.int32, (16,), 0)
        for g in range(8):
            def edge(l, t16):
                j = g * 16 + l
                a16 = jnp.zeros((16,), _F32)
                for q in range(8):
                    sl = pl.ds(q * 16, 16)
                    uv = ub[j, sl] + vb[j, sl]
                    a16 = a16 + _leaky(uv) * attl_v[sl]
                return jnp.where(lane == l, _allsum16(a16), t16)

            t16 = lax.fori_loop(0, 16, edge, jnp.zeros((16,), _F32))
            sl16 = pl.ds(g * 16, 16)
            r16 = plsc.load_gather(rloc, [dstb[sl16]])
            eb[sl16] = jnp.exp(_leaky(t16 + r16))

        pltpu.sync_copy(w_hbm.at[srcb], ub)

        for g in range(8):
            sl16 = pl.ds(g * 16, 16)
            e16 = eb[sl16]

            def scale(l, c2, e16=e16, g=g):
                j = g * 16 + l
                ev = _allsum16(jnp.where(lane == l, e16, 0.0))
                for q in range(8):
                    sl = pl.ds(q * 16, 16)
                    ub[j, sl] = ub[j, sl] * ev
                return c2

            lax.fori_loop(0, 16, scale, 0)

        pltpu.sync_copy(eb, s_sh.at[dstb], add=True)
        pltpu.sync_copy(ub, acc_sh.at[dstb], add=True)
        return carry

    lax.fori_loop(0, nblk, blk, 0)
    plsc.subcore_barrier()

    @pl.when(s < 10)
    def _():
        for k in _CHUNKS:
            hoff = pl.multiple_of(s * _RS + k, 8)
            pltpu.sync_copy(acc_sh.at[pl.ds(hoff, _EB)], ub)
            pltpu.sync_copy(ub, acc_o.at[c, pl.ds(hoff, _EB)])
            soff = pl.multiple_of(c * _N + hoff, 8)
            pltpu.sync_copy(s_sh.at[pl.ds(hoff, _EB)], eb)
            pltpu.sync_copy(eb, s_o.at[pl.ds(soff, _EB)])


def _sc_gate(u, v, w, r, att_l, src, dst):
    mesh = plsc.VectorSubcoreMesh(core_axis_name="c", subcore_axis_name="s")
    fn = pl.kernel(
        _sc_gate_body,
        out_type=[
            jax.ShapeDtypeStruct((_NC, _N, _D), _F32),
            jax.ShapeDtypeStruct((_NC * _N,), _F32),
        ],
        mesh=mesh,
        compiler_params=pltpu.CompilerParams(needs_layout_passes=False),
        scratch_types=[
            pltpu.VMEM((_EB,), jnp.int32),
            pltpu.VMEM((_EB,), jnp.int32),
            pltpu.VMEM((_EB, _D), _F32),
            pltpu.VMEM((_EB, _D), _F32),
            pltpu.VMEM((_EB,), _F32),
            pltpu.VMEM((_N,), _F32),
            pltpu.VMEM((_D,), _F32),
            pltpu.VMEM_SHARED((_N, _D), _F32),
            pltpu.VMEM_SHARED((_N,), _F32),
        ],
    )
    return fn(u, v, w, r, att_l, src, dst)


def _tc_mid_body(accp, sp, h0, g_bias, wih_t, whh_t, bih, bhh,
                 wa_t, asa, ada, xc_o, xw_o, as_o, ad_o):
    acc = accp[0] + accp[1]
    seg = sp[0] + sp[1]
    h = _elu(acc / (seg + 1e-16) + g_bias[...])
    xc = jnp.maximum(_gru(h, h0[...], wih_t[...], whh_t[...],
                          bih[...], bhh[...]), 0.0)
    xw = _dot(xc, wa_t[...])
    xc_o[...] = xc
    xw_o[...] = xw
    as_o[...] = jnp.sum(xw * asa[...][None, :], axis=1, keepdims=True)
    ad_o[...] = jnp.sum(xw * ada[...][None, :], axis=1, keepdims=True)


def _tc_mid(accp, sp, h0, g_bias, wih_t, whh_t, bih, bhh, wa_t, asa, ada):
    nb = 10
    rows = _N // nb
    full = lambda s: pl.BlockSpec(s, lambda i: (0,) * len(s))
    return pl.pallas_call(
        _tc_mid_body,
        grid=(nb,),
        in_specs=[
            pl.BlockSpec((_NC, rows, _D), lambda i: (0, i, 0)),
            pl.BlockSpec((_NC, rows, 1), lambda i: (0, i, 0)),
            pl.BlockSpec((rows, _D), lambda i: (i, 0)),
            full((_D,)),
            full((_D, 3 * _D)), full((_D, 3 * _D)),
            full((3 * _D,)), full((3 * _D,)),
            full((_D, _D)), full((_D,)), full((_D,)),
        ],
        out_specs=[
            pl.BlockSpec((rows, _D), lambda i: (i, 0)),
            pl.BlockSpec((rows, _D), lambda i: (i, 0)),
            pl.BlockSpec((rows, 1), lambda i: (i, 0)),
            pl.BlockSpec((rows, 1), lambda i: (i, 0)),
        ],
        out_shape=[
            jax.ShapeDtypeStruct((_N, _D), _F32),
            jax.ShapeDtypeStruct((_N, _D), _F32),
            jax.ShapeDtypeStruct((_N, 1), _F32),
            jax.ShapeDtypeStruct((_N, 1), _F32),
        ],
    )(accp, sp, h0, g_bias, wih_t, whh_t, bih, bhh, wa_t, asa, ada)


def _sc_gat_body(xw_hbm, as_hbm, ad_hbm, src_hbm, dst_hbm,
                 acc_o, s_o,
                 srcb, dstb, xb, eb, asloc, adloc,
                 acc_sh, s_sh):
    c = lax.axis_index("c")
    s = lax.axis_index("s")
    wid = s * _NC + c
    def zrow(j, c2):
        for q in range(8):
            xb[j, pl.ds(q * 16, 16)] = jnp.zeros((16,), _F32)
        return c2

    lax.fori_loop(0, _EB, zrow, 0)
    for q in range(8):
        eb[pl.ds(q * 16, 16)] = jnp.zeros((16,), _F32)
    for k in range(5):
        pltpu.sync_copy(xb.at[pl.ds(0, 125)],
                        acc_sh.at[pl.ds(s * 625 + k * 125, 125)])

    @pl.when(s < 10)
    def _():
        for k in _CHUNKS:
            pltpu.sync_copy(eb, s_sh.at[pl.ds(s * _RS + k, _EB)])

    pltpu.sync_copy(as_hbm, asloc)
    pltpu.sync_copy(ad_hbm, adloc)
    plsc.subcore_barrier()

    nblk = _NBLK // _NW + jnp.where(wid < _NBLK % _NW, 1, 0)

    def blk(i, carry):
        base = (wid + i * _NW) * _EB
        base = pl.multiple_of(base, _EB)
        pltpu.sync_copy(src_hbm.at[pl.ds(base, _EB)], srcb)
        pltpu.sync_copy(dst_hbm.at[pl.ds(base, _EB)], dstb)
        pltpu.sync_copy(xw_hbm.at[srcb], xb)

        lane = lax.broadcasted_iota(jnp.int32, (16,), 0)
        for g in range(8):
            sl16 = pl.ds(g * 16, 16)
            a16 = plsc.load_gather(asloc, [srcb[sl16]])
            d16 = plsc.load_gather(adloc, [dstb[sl16]])
            e16 = jnp.exp(_leaky(a16 + d16))
            eb[sl16] = e16

            def scale(l, c2, e16=e16, g=g):
                j = g * 16 + l
                ev = _allsum16(jnp.where(lane == l, e16, 0.0))
                for q in range(8):
                    sl = pl.ds(q * 16, 16)
                    xb[j, sl] = xb[j, sl] * ev
                return c2

            lax.fori_loop(0, 16, scale, 0)

        pltpu.sync_copy(eb, s_sh.at[dstb], add=True)
        pltpu.sync_copy(xb, acc_sh.at[dstb], add=True)
        return carry

    lax.fori_loop(0, nblk, blk, 0)
    plsc.subcore_barrier()

    @pl.when(s < 10)
    def _():
        for k in _CHUNKS:
            hoff = pl.multiple_of(s * _RS + k, 8)
            pltpu.sync_copy(acc_sh.at[pl.ds(hoff, _EB)], xb)
            pltpu.sync_copy(xb, acc_o.at[c, pl.ds(hoff, _EB)])
            soff = pl.multiple_of(c * _N + hoff, 8)
            pltpu.sync_copy(s_sh.at[pl.ds(hoff, _EB)], eb)
            pltpu.sync_copy(eb, s_o.at[pl.ds(soff, _EB)])


def _sc_gat(xw, a_s, a_d, src, dst):
    mesh = plsc.VectorSubcoreMesh(core_axis_name="c", subcore_axis_name="s")
    fn = pl.kernel(
        _sc_gat_body,
        out_type=[
            jax.ShapeDtypeStruct((_NC, _N, _D), _F32),
            jax.ShapeDtypeStruct((_NC * _N,), _F32),
        ],
        mesh=mesh,
        compiler_params=pltpu.CompilerParams(needs_layout_passes=False),
        scratch_types=[
            pltpu.VMEM((_EB,), jnp.int32),
            pltpu.VMEM((_EB,), jnp.int32),
            pltpu.VMEM((_EB, _D), _F32),
            pltpu.VMEM((_EB,), _F32),
            pltpu.VMEM((_N,), _F32),
            pltpu.VMEM((_N,), _F32),
            pltpu.VMEM_SHARED((_N, _D), _F32),
            pltpu.VMEM_SHARED((_N,), _F32),
        ],
    )
    return fn(xw, a_s, a_d, src, dst)


def _tc_mid2_body(accp, sp, xc, bias_a, wih_t, whh_t, bih, bhh, xc2_o):
    acc = accp[0] + accp[1]
    seg = sp[0] + sp[1]
    h = _elu(acc / (seg + 1e-16) + bias_a[...])
    xc2_o[...] = jnp.maximum(
        _gru(h, xc[...], wih_t[...], whh_t[...], bih[...], bhh[...]), 0.0)


def _tc_mid2(accp, sp, xc, bias_a, wih_t, whh_t, bih, bhh):
    nb = 10
    rows = _N // nb
    full = lambda s: pl.BlockSpec(s, lambda i: (0,) * len(s))
    return pl.pallas_call(
        _tc_mid2_body,
        grid=(nb,),
        in_specs=[
            pl.BlockSpec((_NC, rows, _D), lambda i: (0, i, 0)),
            pl.BlockSpec((_NC, rows, 1), lambda i: (0, i, 0)),
            pl.BlockSpec((rows, _D), lambda i: (i, 0)),
            full((_D,)),
            full((_D, 3 * _D)), full((_D, 3 * _D)),
            full((3 * _D,)), full((3 * _D,)),
        ],
        out_specs=pl.BlockSpec((rows, _D), lambda i: (i, 0)),
        out_shape=jax.ShapeDtypeStruct((_N, _D), _F32),
    )(accp, sp, xc, bias_a, wih_t, whh_t, bih, bhh)


def _tc_final_body(xc2_r, batch_r, wm_t, attsm, attdm, bias_m,
                   wih_t, whh_t, bih, bhh, wout_t, b_out,
                   mol, wmol_t, bmol, cont,
                   g0, b0, g1, b1, g2, b2,
                   p0a_t, p0b_t, p0c_t, p0_b, p1_t, p1_b, p2_t, p2_b,
                   y_o):
    xc2 = xc2_r[...]
    batch = batch_r[...]
    onehot = (lax.broadcasted_iota(jnp.int32, (_B, _N), 0)
              == batch[None, :]).astype(_F32)
    out = jnp.maximum(_dot_hi(onehot, xc2), 0.0)
    for _ in range(2):
        xs = _dot(xc2, wm_t[...])
        xd = _dot(out, wm_t[...])
        a_sn = jnp.sum(xs * attsm[...][None, :], axis=1)
        a_dm = jnp.sum(xd * attdm[...][None, :], axis=1)
        adb = jnp.sum(onehot * a_dm[:, None], axis=0)
        al = _leaky(a_sn + adb)
        masked = jnp.where(onehot > 0, al[None, :], -1e30)
        m = jnp.max(masked, axis=1)
        m = jnp.where(m > -1e29, m, 0.0)
        mb = jnp.sum(onehot * m[:, None], axis=0)
        e = jnp.exp(al - mb)
        sseg = jnp.sum(onehot * e[None, :], axis=1)
        sb = jnp.sum(onehot * sseg[:, None], axis=0)
        al2 = e / (sb + 1e-16)
        hm = _elu(_dot_hi(onehot, xs * al2[:, None]) + bias_m[...])
        out = jnp.maximum(_gru(hm, out, wih_t[...], whh_t[...],
                               bih[...], bhh[...]), 0.0)
    afp = _dot(out, wout_t[...]) + b_out[...]
    me = jnp.maximum(_dot(mol[...], wmol_t[...]) + bmol[...], 0.0)

    def bn(v, g, b):
        mu = jnp.mean(v, axis=0)
        var = jnp.mean((v - mu[None, :]) ** 2, axis=0)
        return (v - mu[None, :]) / jnp.sqrt(var[None, :] + 1e-5) * g[...] + b[...]

    z = (_dot(bn(afp, g0, b0), p0a_t[...])
         + _dot(bn(me, g1, b1), p0b_t[...])
         + _dot(bn(cont[...], g2, b2), p0c_t[...]) + p0_b[...])
    y = jnp.maximum(z, 0.0)
    y = jnp.maximum(_dot(y, p1_t[...]) + p1_b[...], 0.0)
    y_o[...] = _dot(y, p2_t[...]) + p2_b[...]


def _tc_final(xc2, batch, args):
    return pl.pallas_call(
        _tc_final_body,
        out_shape=jax.ShapeDtypeStruct((_B, 1), _F32),
    )(xc2, batch, *args)


@jax.jit
def kernel(x, edge_attr, mol_attr, mlogP, volumn, wt, params, edge_index,
           batch):
    p = params
    src = edge_index[0]
    dst = edge_index[1]
    cont = jnp.stack([mlogP, volumn, wt], axis=1)

    h0, u, w, r = _tc_nodes(
        x, p['W_node'].T, p['b_node'], p['W_lin1'].T, p['b_lin1'],
        p['g_lin1'][:, :_D].T, p['g_lin2'].T, p['att_r'])
    v = _tc_edges(edge_attr, p['W_edge'].T, p['b_edge'],
                  p['g_lin1'][:, _D:].T)
    accp, sp = _sc_gate(u, v, w, r.reshape(_N), p['att_l'], src, dst)
    xc, xw, a_s, a_d = _tc_mid(
        accp, sp.reshape(_NC, _N, 1), h0, p['g_bias'], p['gru1_Wih'].T,
        p['gru1_Whh'].T, p['gru1_bih'], p['gru1_bhh'], p['Wa'].T,
        p['att_src_a'], p['att_dst_a'])
    accp2, sp2 = _sc_gat(xw, a_s.reshape(_N), a_d.reshape(_N), src, dst)
    xc2 = _tc_mid2(accp2, sp2.reshape(_NC, _N, 1), xc, p['bias_a'], p['grua_Wih'].T,
                   p['grua_Whh'].T, p['grua_bih'], p['grua_bhh'])
    y = _tc_final(xc2, batch, (
        p['Wm'].T, p['att_src_m'], p['att_dst_m'], p['bias_m'],
        p['grum_Wih'].T, p['grum_Whh'].T, p['grum_bih'], p['grum_bhh'],
        p['W_out'].T, p['b_out'],
        mol_attr, p['W_mol'].T, p['b_mol'], cont,
        p['bn_gamma'][:64], p['bn_beta'][:64],
        p['bn_gamma'][64:96], p['bn_beta'][64:96],
        p['bn_gamma'][96:], p['bn_beta'][96:],
        p['P0_W'][:, :64].T, p['P0_W'][:, 64:96].T, p['P0_W'][:, 96:].T,
        p['P0_b'], p['P1_W'].T, p['P1_b'], p['P2_W'].T, p['P2_b']))
    return y


# speedup vs baseline: 14.2313x; 1.1382x over previous
import functools
import jax
import jax.numpy as jnp
from jax import lax
from jax.experimental import pallas as pl
from jax.experimental.pallas import tpu as pltpu
from jax.experimental.pallas import tpu_sc as plsc

_N = 10000
_E = 320000
_B = 64
_D = 128
_F32 = jnp.float32

_NC, _NS = 2, 16
_NW = _NC * _NS
_EB = 128
_NBLK = _E // _EB
_RT = _N // _NS
_RS = _N // 10
_CHUNKS = (0, 128, 256, 384, 512, 640, 768, 872)


def _leaky(x):
    return jnp.where(x >= 0, x, 0.01 * x)


def _allsum16(x):
    lane = lax.broadcasted_iota(jnp.int32, (16,), 0)
    for k in (8, 4, 2, 1):
        perm = jnp.bitwise_xor(lane, k)
        x = x + x.at[perm].get(mode="promise_in_bounds")
    return x


def _elu(x):
    return jnp.where(x > 0, x, jnp.exp(jnp.minimum(x, 0.0)) - 1.0)


def _dot(a, b):
    return jnp.dot(a, b, preferred_element_type=_F32)


def _dot_hi(a, b):
    return jnp.dot(a, b, preferred_element_type=_F32,
                   precision=lax.Precision.HIGHEST)


def _gru(xin, h, wih_t, whh_t, bih, bhh):
    gi = _dot(xin, wih_t) + bih
    gh = _dot(h, whh_t) + bhh
    r = jax.nn.sigmoid(gi[:, :_D] + gh[:, :_D])
    z = jax.nn.sigmoid(gi[:, _D:2 * _D] + gh[:, _D:2 * _D])
    n = jnp.tanh(gi[:, 2 * _D:] + r * gh[:, 2 * _D:])
    return (1.0 - z) * n + z * h


def _tc_nodes_body(x, wn_t, bn, wl1_t, bl1, a_t, g2_t, att_r,
                   h0_o, u_o, w_o, r_o):
    xe = jnp.maximum(_dot(x[...], wn_t[...]) + bn[...], 0.0)
    h0 = _leaky(_dot(xe, wl1_t[...]) + bl1[...])
    h0_o[...] = h0
    u_o[...] = _dot(h0, a_t[...])
    w_o[...] = _dot(h0, g2_t[...])
    r_o[...] = jnp.sum(h0 * att_r[...][None, :], axis=1, keepdims=True)


def _tc_nodes(x, wn_t, bn, wl1_t, bl1, a_t, g2_t, att_r):
    nb = 10
    rows = _N // nb
    full = lambda s: pl.BlockSpec(s, lambda i: (0,) * len(s))
    return pl.pallas_call(
        _tc_nodes_body,
        grid=(nb,),
        in_specs=[
            pl.BlockSpec((rows, _D), lambda i: (i, 0)),
            full((_D, _D)), full((_D,)), full((_D, _D)), full((_D,)),
            full((_D, _D)), full((_D, _D)), full((_D,)),
        ],
        out_specs=[
            pl.BlockSpec((rows, _D), lambda i: (i, 0)),
            pl.BlockSpec((rows, _D), lambda i: (i, 0)),
            pl.BlockSpec((rows, _D), lambda i: (i, 0)),
            pl.BlockSpec((rows, 1), lambda i: (i, 0)),
        ],
        out_shape=[
            jax.ShapeDtypeStruct((_N, _D), _F32),
            jax.ShapeDtypeStruct((_N, _D), _F32),
            jax.ShapeDtypeStruct((_N, _D), _F32),
            jax.ShapeDtypeStruct((_N, 1), _F32),
        ],
    )(x, wn_t, bn, wl1_t, bl1, a_t, g2_t, att_r)


def _tc_edges_body(ea, we_t, be, b_t, v_o):
    ee = jnp.maximum(_dot(ea[...], we_t[...]) + be[...], 0.0)
    v_o[...] = _dot(ee, b_t[...])


def _tc_edges(edge_attr, we_t, be, b_t):
    nb = 160
    rows = _E // nb
    full = lambda s: pl.BlockSpec(s, lambda i: (0,) * len(s))
    return pl.pallas_call(
        _tc_edges_body,
        grid=(nb,),
        in_specs=[
            pl.BlockSpec((rows, 16), lambda i: (i, 0)),
            full((16, 16)), full((16,)), full((16, _D)),
        ],
        out_specs=pl.BlockSpec((rows, _D), lambda i: (i, 0)),
        out_shape=jax.ShapeDtypeStruct((_E, _D), _F32),
    )(edge_attr, we_t, be, b_t)


def _sc_gate_body(u_hbm, v_hbm, w_hbm, r_hbm, attl_hbm, src_hbm, dst_hbm,
                  acc_o, s_o,
                  srcb, dstb, ub, vb, wb, eb, rb, attl_v,
                  sem_a, sem_b, sem_c,
                  acc_sh, s_sh):
    c = lax.axis_index("c")
    s = lax.axis_index("s")
    wid = s * _NC + c
    def zrow(j, c2):
        for q in range(8):
            ub[j, pl.ds(q * 16, 16)] = jnp.zeros((16,), _F32)
        return c2

    lax.fori_loop(0, _EB, zrow, 0)
    for q in range(8):
        eb[pl.ds(q * 16, 16)] = jnp.zeros((16,), _F32)
    for k in range(5):
        pltpu.sync_copy(ub.at[pl.ds(0, 125)],
                        acc_sh.at[pl.ds(s * 625 + k * 125, 125)])

    @pl.when(s < 10)
    def _():
        for k in _CHUNKS:
            pltpu.sync_copy(eb, s_sh.at[pl.ds(s * _RS + k, _EB)])

    pltpu.sync_copy(attl_hbm, attl_v)
    plsc.subcore_barrier()

    nblk = _NBLK // _NW + jnp.where(wid < _NBLK % _NW, 1, 0)

    def blk(i, carry):
        base = (wid + i * _NW) * _EB
        base = pl.multiple_of(base, _EB)
        d1 = pltpu.async_copy(src_hbm.at[pl.ds(base, _EB)], srcb, sem_a)
        d2 = pltpu.async_copy(dst_hbm.at[pl.ds(base, _EB)], dstb, sem_a)
        d3 = pltpu.async_copy(v_hbm.at[pl.ds(base, _EB)], vb, sem_a)
        d1.wait(); d2.wait(); d3.wait()
        g1 = pltpu.async_copy(u_hbm.at[srcb], ub, sem_b)
        g2 = pltpu.async_copy(w_hbm.at[srcb], wb, sem_b)
        g3 = pltpu.async_copy(r_hbm.at[dstb], rb, sem_b)
        g1.wait(); g2.wait(); g3.wait()

        lane = lax.broadcasted_iota(jnp.int32, (16,), 0)
        for g in range(8):
            def edge(l, t16):
                j = g * 16 + l
                a16 = jnp.zeros((16,), _F32)
                for q in range(8):
                    sl = pl.ds(q * 16, 16)
                    uv = ub[j, sl] + vb[j, sl]
                    a16 = a16 + _leaky(uv) * attl_v[sl]
                return jnp.where(lane == l, _allsum16(a16), t16)

            t16 = lax.fori_loop(0, 16, edge, jnp.zeros((16,), _F32))
            sl16 = pl.ds(g * 16, 16)
            eb[sl16] = jnp.exp(_leaky(t16 + rb[sl16]))

        for g in range(8):
            sl16 = pl.ds(g * 16, 16)
            e16 = eb[sl16]

            def scale(l, c2, e16=e16, g=g):
                j = g * 16 + l
                ev = _allsum16(jnp.where(lane == l, e16, 0.0))
                for q in range(8):
                    sl = pl.ds(q * 16, 16)
                    wb[j, sl] = wb[j, sl] * ev
                return c2

            lax.fori_loop(0, 16, scale, 0)

        c1 = pltpu.async_copy(eb, s_sh.at[dstb], sem_c, add=True)
        c2d = pltpu.async_copy(wb, acc_sh.at[dstb], sem_c, add=True)
        c1.wait(); c2d.wait()
        return carry

    lax.fori_loop(0, nblk, blk, 0)
    plsc.subcore_barrier()

    @pl.when(s < 10)
    def _():
        for k in _CHUNKS:
            hoff = pl.multiple_of(s * _RS + k, 8)
            pltpu.sync_copy(acc_sh.at[pl.ds(hoff, _EB)], ub)
            pltpu.sync_copy(ub, acc_o.at[c, pl.ds(hoff, _EB)])
            soff = pl.multiple_of(c * _N + hoff, 8)
            pltpu.sync_copy(s_sh.at[pl.ds(hoff, _EB)], eb)
            pltpu.sync_copy(eb, s_o.at[pl.ds(soff, _EB)])


def _sc_gate(u, v, w, r, att_l, src, dst):
    mesh = plsc.VectorSubcoreMesh(core_axis_name="c", subcore_axis_name="s")
    fn = pl.kernel(
        _sc_gate_body,
        out_type=[
            jax.ShapeDtypeStruct((_NC, _N, _D), _F32),
            jax.ShapeDtypeStruct((_NC * _N,), _F32),
        ],
        mesh=mesh,
        compiler_params=pltpu.CompilerParams(needs_layout_passes=False),
        scratch_types=[
            pltpu.VMEM((_EB,), jnp.int32),
            pltpu.VMEM((_EB,), jnp.int32),
            pltpu.VMEM((_EB, _D), _F32),
            pltpu.VMEM((_EB, _D), _F32),
            pltpu.VMEM((_EB, _D), _F32),
            pltpu.VMEM((_EB,), _F32),
            pltpu.VMEM((_EB,), _F32),
            pltpu.VMEM((_D,), _F32),
            pltpu.SemaphoreType.DMA,
            pltpu.SemaphoreType.DMA,
            pltpu.SemaphoreType.DMA,
            pltpu.VMEM_SHARED((_N, _D), _F32),
            pltpu.VMEM_SHARED((_N,), _F32),
        ],
    )
    return fn(u, v, w, r, att_l, src, dst)


def _tc_mid_body(accp, sp, h0, g_bias, wih_t, whh_t, bih, bhh,
                 wa_t, asa, ada, xc_o, xw_o, as_o, ad_o):
    acc = accp[0] + accp[1]
    seg = sp[0] + sp[1]
    h = _elu(acc / (seg + 1e-16) + g_bias[...])
    xc = jnp.maximum(_gru(h, h0[...], wih_t[...], whh_t[...],
                          bih[...], bhh[...]), 0.0)
    xw = _dot(xc, wa_t[...])
    xc_o[...] = xc
    xw_o[...] = xw
    as_o[...] = jnp.sum(xw * asa[...][None, :], axis=1, keepdims=True)
    ad_o[...] = jnp.sum(xw * ada[...][None, :], axis=1, keepdims=True)


def _tc_mid(accp, sp, h0, g_bias, wih_t, whh_t, bih, bhh, wa_t, asa, ada):
    nb = 10
    rows = _N // nb
    full = lambda s: pl.BlockSpec(s, lambda i: (0,) * len(s))
    return pl.pallas_call(
        _tc_mid_body,
        grid=(nb,),
        in_specs=[
            pl.BlockSpec((_NC, rows, _D), lambda i: (0, i, 0)),
            pl.BlockSpec((_NC, rows, 1), lambda i: (0, i, 0)),
            pl.BlockSpec((rows, _D), lambda i: (i, 0)),
            full((_D,)),
            full((_D, 3 * _D)), full((_D, 3 * _D)),
            full((3 * _D,)), full((3 * _D,)),
            full((_D, _D)), full((_D,)), full((_D,)),
        ],
        out_specs=[
            pl.BlockSpec((rows, _D), lambda i: (i, 0)),
            pl.BlockSpec((rows, _D), lambda i: (i, 0)),
            pl.BlockSpec((rows, 1), lambda i: (i, 0)),
            pl.BlockSpec((rows, 1), lambda i: (i, 0)),
        ],
        out_shape=[
            jax.ShapeDtypeStruct((_N, _D), _F32),
            jax.ShapeDtypeStruct((_N, _D), _F32),
            jax.ShapeDtypeStruct((_N, 1), _F32),
            jax.ShapeDtypeStruct((_N, 1), _F32),
        ],
    )(accp, sp, h0, g_bias, wih_t, whh_t, bih, bhh, wa_t, asa, ada)


def _sc_gat_body(xw_hbm, as_hbm, ad_hbm, src_hbm, dst_hbm,
                 acc_o, s_o,
                 srcb, dstb, xb, eb, asloc, adloc,
                 sem_a, sem_b, sem_c,
                 acc_sh, s_sh):
    c = lax.axis_index("c")
    s = lax.axis_index("s")
    wid = s * _NC + c
    def zrow(j, c2):
        for q in range(8):
            xb[j, pl.ds(q * 16, 16)] = jnp.zeros((16,), _F32)
        return c2

    lax.fori_loop(0, _EB, zrow, 0)
    for q in range(8):
        eb[pl.ds(q * 16, 16)] = jnp.zeros((16,), _F32)
    for k in range(5):
        pltpu.sync_copy(xb.at[pl.ds(0, 125)],
                        acc_sh.at[pl.ds(s * 625 + k * 125, 125)])

    @pl.when(s < 10)
    def _():
        for k in _CHUNKS:
            pltpu.sync_copy(eb, s_sh.at[pl.ds(s * _RS + k, _EB)])

    pltpu.sync_copy(as_hbm, asloc)
    pltpu.sync_copy(ad_hbm, adloc)
    plsc.subcore_barrier()

    nblk = _NBLK // _NW + jnp.where(wid < _NBLK % _NW, 1, 0)

    def blk(i, carry):
        base = (wid + i * _NW) * _EB
        base = pl.multiple_of(base, _EB)
        d1 = pltpu.async_copy(src_hbm.at[pl.ds(base, _EB)], srcb, sem_a)
        d2 = pltpu.async_copy(dst_hbm.at[pl.ds(base, _EB)], dstb, sem_a)
        d1.wait(); d2.wait()
        gx = pltpu.async_copy(xw_hbm.at[srcb], xb, sem_b)

        lane = lax.broadcasted_iota(jnp.int32, (16,), 0)
        for g in range(8):
            sl16 = pl.ds(g * 16, 16)
            a16 = plsc.load_gather(asloc, [srcb[sl16]])
            d16 = plsc.load_gather(adloc, [dstb[sl16]])
            e16 = jnp.exp(_leaky(a16 + d16))
            eb[sl16] = e16

        gx.wait()
        for g in range(8):
            sl16 = pl.ds(g * 16, 16)
            e16 = eb[sl16]

            def scale(l, c2, e16=e16, g=g):
                j = g * 16 + l
                ev = _allsum16(jnp.where(lane == l, e16, 0.0))
                for q in range(8):
                    sl = pl.ds(q * 16, 16)
                    xb[j, sl] = xb[j, sl] * ev
                return c2

            lax.fori_loop(0, 16, scale, 0)

        c1 = pltpu.async_copy(eb, s_sh.at[dstb], sem_c, add=True)
        c2d = pltpu.async_copy(xb, acc_sh.at[dstb], sem_c, add=True)
        c1.wait(); c2d.wait()
        return carry

    lax.fori_loop(0, nblk, blk, 0)
    plsc.subcore_barrier()

    @pl.when(s < 10)
    def _():
        for k in _CHUNKS:
            hoff = pl.multiple_of(s * _RS + k, 8)
            pltpu.sync_copy(acc_sh.at[pl.ds(hoff, _EB)], xb)
            pltpu.sync_copy(xb, acc_o.at[c, pl.ds(hoff, _EB)])
            soff = pl.multiple_of(c * _N + hoff, 8)
            pltpu.sync_copy(s_sh.at[pl.ds(hoff, _EB)], eb)
            pltpu.sync_copy(eb, s_o.at[pl.ds(soff, _EB)])


def _sc_gat(xw, a_s, a_d, src, dst):
    mesh = plsc.VectorSubcoreMesh(core_axis_name="c", subcore_axis_name="s")
    fn = pl.kernel(
        _sc_gat_body,
        out_type=[
            jax.ShapeDtypeStruct((_NC, _N, _D), _F32),
            jax.ShapeDtypeStruct((_NC * _N,), _F32),
        ],
        mesh=mesh,
        compiler_params=pltpu.CompilerParams(needs_layout_passes=False),
        scratch_types=[
            pltpu.VMEM((_EB,), jnp.int32),
            pltpu.VMEM((_EB,), jnp.int32),
            pltpu.VMEM((_EB, _D), _F32),
            pltpu.VMEM((_EB,), _F32),
            pltpu.VMEM((_N,), _F32),
            pltpu.VMEM((_N,), _F32),
            pltpu.SemaphoreType.DMA,
            pltpu.SemaphoreType.DMA,
            pltpu.SemaphoreType.DMA,
            pltpu.VMEM_SHARED((_N, _D), _F32),
            pltpu.VMEM_SHARED((_N,), _F32),
        ],
    )
    return fn(xw, a_s, a_d, src, dst)


def _tc_mid2_body(accp, sp, xc, bias_a, wih_t, whh_t, bih, bhh, xc2_o):
    acc = accp[0] + accp[1]
    seg = sp[0] + sp[1]
    h = _elu(acc / (seg + 1e-16) + bias_a[...])
    xc2_o[...] = jnp.maximum(
        _gru(h, xc[...], wih_t[...], whh_t[...], bih[...], bhh[...]), 0.0)


def _tc_mid2(accp, sp, xc, bias_a, wih_t, whh_t, bih, bhh):
    nb = 10
    rows = _N // nb
    full = lambda s: pl.BlockSpec(s, lambda i: (0,) * len(s))
    return pl.pallas_call(
        _tc_mid2_body,
        grid=(nb,),
        in_specs=[
            pl.BlockSpec((_NC, rows, _D), lambda i: (0, i, 0)),
            pl.BlockSpec((_NC, rows, 1), lambda i: (0, i, 0)),
            pl.BlockSpec((rows, _D), lambda i: (i, 0)),
            full((_D,)),
            full((_D, 3 * _D)), full((_D, 3 * _D)),
            full((3 * _D,)), full((3 * _D,)),
        ],
        out_specs=pl.BlockSpec((rows, _D), lambda i: (i, 0)),
        out_shape=jax.ShapeDtypeStruct((_N, _D), _F32),
    )(accp, sp, xc, bias_a, wih_t, whh_t, bih, bhh)


def _tc_final_body(xc2_r, batch_r, wm_t, attsm, attdm, bias_m,
                   wih_t, whh_t, bih, bhh, wout_t, b_out,
                   mol, wmol_t, bmol, cont,
                   g0, b0, g1, b1, g2, b2,
                   p0a_t, p0b_t, p0c_t, p0_b, p1_t, p1_b, p2_t, p2_b,
                   y_o):
    xc2 = xc2_r[...]
    batch = batch_r[...]
    onehot = (lax.broadcasted_iota(jnp.int32, (_B, _N), 0)
              == batch[None, :]).astype(_F32)
    out = jnp.maximum(_dot_hi(onehot, xc2), 0.0)
    for _ in range(2):
        xs = _dot(xc2, wm_t[...])
        xd = _dot(out, wm_t[...])
        a_sn = jnp.sum(xs * attsm[...][None, :], axis=1)
        a_dm = jnp.sum(xd * attdm[...][None, :], axis=1)
        adb = jnp.sum(onehot * a_dm[:, None], axis=0)
        al = _leaky(a_sn + adb)
        masked = jnp.where(onehot > 0, al[None, :], -1e30)
        m = jnp.max(masked, axis=1)
        m = jnp.where(m > -1e29, m, 0.0)
        mb = jnp.sum(onehot * m[:, None], axis=0)
        e = jnp.exp(al - mb)
        sseg = jnp.sum(onehot * e[None, :], axis=1)
        sb = jnp.sum(onehot * sseg[:, None], axis=0)
        al2 = e / (sb + 1e-16)
        hm = _elu(_dot_hi(onehot, xs * al2[:, None]) + bias_m[...])
        out = jnp.maximum(_gru(hm, out, wih_t[...], whh_t[...],
                               bih[...], bhh[...]), 0.0)
    afp = _dot(out, wout_t[...]) + b_out[...]
    me = jnp.maximum(_dot(mol[...], wmol_t[...]) + bmol[...], 0.0)

    def bn(v, g, b):
        mu = jnp.mean(v, axis=0)
        var = jnp.mean((v - mu[None, :]) ** 2, axis=0)
        return (v - mu[None, :]) / jnp.sqrt(var[None, :] + 1e-5) * g[...] + b[...]

    z = (_dot(bn(afp, g0, b0), p0a_t[...])
         + _dot(bn(me, g1, b1), p0b_t[...])
         + _dot(bn(cont[...], g2, b2), p0c_t[...]) + p0_b[...])
    y = jnp.maximum(z, 0.0)
    y = jnp.maximum(_dot(y, p1_t[...]) + p1_b[...], 0.0)
    y_o[...] = _dot(y, p2_t[...]) + p2_b[...]


def _tc_final(xc2, batch, args):
    return pl.pallas_call(
        _tc_final_body,
        out_shape=jax.ShapeDtypeStruct((_B, 1), _F32),
    )(xc2, batch, *args)


@jax.jit
def kernel(x, edge_attr, mol_attr, mlogP, volumn, wt, params, edge_index,
           batch):
    p = params
    src = edge_index[0]
    dst = edge_index[1]
    cont = jnp.stack([mlogP, volumn, wt], axis=1)

    h0, u, w, r = _tc_nodes(
        x, p['W_node'].T, p['b_node'], p['W_lin1'].T, p['b_lin1'],
        p['g_lin1'][:, :_D].T, p['g_lin2'].T, p['att_r'])
    v = _tc_edges(edge_attr, p['W_edge'].T, p['b_edge'],
                  p['g_lin1'][:, _D:].T)
    accp, sp = _sc_gate(u, v, w, r.reshape(_N), p['att_l'], src, dst)
    xc, xw, a_s, a_d = _tc_mid(
        accp, sp.reshape(_NC, _N, 1), h0, p['g_bias'], p['gru1_Wih'].T,
        p['gru1_Whh'].T, p['gru1_bih'], p['gru1_bhh'], p['Wa'].T,
        p['att_src_a'], p['att_dst_a'])
    accp2, sp2 = _sc_gat(xw, a_s.reshape(_N), a_d.reshape(_N), src, dst)
    xc2 = _tc_mid2(accp2, sp2.reshape(_NC, _N, 1), xc, p['bias_a'], p['grua_Wih'].T,
                   p['grua_Whh'].T, p['grua_bih'], p['grua_bhh'])
    y = _tc_final(xc2, batch, (
        p['Wm'].T, p['att_src_m'], p['att_dst_m'], p['bias_m'],
        p['grum_Wih'].T, p['grum_Whh'].T, p['grum_bih'], p['grum_bhh'],
        p['W_out'].T, p['b_out'],
        mol_attr, p['W_mol'].T, p['b_mol'], cont,
        p['bn_gamma'][:64], p['bn_beta'][:64],
        p['bn_gamma'][64:96], p['bn_beta'][64:96],
        p['bn_gamma'][96:], p['bn_beta'][96:],
        p['P0_W'][:, :64].T, p['P0_W'][:, 64:96].T, p['P0_W'][:, 96:].T,
        p['P0_b'], p['P1_W'].T, p['P1_b'], p['P2_W'].T, p['P2_b']))
    return y
